# Initial kernel scaffold; baseline (speedup 1.0000x reference)
import jax
import jax.numpy as jnp
from jax import lax
from jax.experimental import pallas as pl
from jax.experimental.pallas import tpu as pltpu
from jax.experimental.pallas import tpu_sc as plsc

N = 10000
E = 320000
H = 8
C1 = 16
C2 = 64
NCORE = 2
NSUB = 16
LANES = 16
E_PER_SC = E // NCORE
E_PER_TILE = E_PER_SC // NSUB
CK = 80
NCHUNK = E_PER_TILE // CK
RPT = N // NSUB
RB = 125

_f32 = jnp.float32


_TCR = 400
_TCG = N // _TCR


def _mm_score_body(x_ref, w_ref, a_ref, xw_ref, sc_ref):
    xw = jnp.dot(x_ref[...], w_ref[...], preferred_element_type=_f32)
    xw_ref[...] = xw
    sc_ref[...] = jnp.dot(xw, a_ref[...], preferred_element_type=_f32)


def _tc_mm_score(x, W, A):
    f_in = x.shape[1]
    f_out = W.shape[1]
    return pl.pallas_call(
        _mm_score_body,
        grid=(_TCG,),
        in_specs=[
            pl.BlockSpec((_TCR, f_in), lambda i: (i, 0)),
            pl.BlockSpec((f_in, f_out), lambda i: (0, 0)),
            pl.BlockSpec((f_out, 16), lambda i: (0, 0)),
        ],
        out_specs=[
            pl.BlockSpec((_TCR, f_out), lambda i: (i, 0)),
            pl.BlockSpec((_TCR, 16), lambda i: (i, 0)),
        ],
        out_shape=[
            jax.ShapeDtypeStruct((N, f_out), _f32),
            jax.ShapeDtypeStruct((N, 16), _f32),
        ],
    )(x, W, A)


def _mid_body(acc_ref, den_ref, b1_ref, w2_ref, a2_ref, exp1_ref,
              xw2_ref, sc2_ref):
    accs = acc_ref[0] + acc_ref[1]
    dens = den_ref[0] + den_ref[1]
    denx = jnp.dot(dens, exp1_ref[...], preferred_element_type=_f32)
    h1 = accs / (denx + 1e-16) + b1_ref[...]
    h1 = jnp.where(h1 > 0, h1, jnp.exp(jnp.minimum(h1, 0.0)) - 1.0)
    xw2 = jnp.dot(h1, w2_ref[...], preferred_element_type=_f32)
    xw2_ref[...] = xw2
    sc2_ref[...] = jnp.dot(xw2, a2_ref[...], preferred_element_type=_f32)


def _tc_mid(acc1p, den1p, b1, W2, A2, Exp1):
    return pl.pallas_call(
        _mid_body,
        grid=(_TCG,),
        in_specs=[
            pl.BlockSpec((2, _TCR, 128), lambda i: (0, i, 0)),
            pl.BlockSpec((2, _TCR, 16), lambda i: (0, i, 0)),
            pl.BlockSpec((1, 128), lambda i: (0, 0)),
            pl.BlockSpec((128, 512), lambda i: (0, 0)),
            pl.BlockSpec((512, 16), lambda i: (0, 0)),
            pl.BlockSpec((16, 128), lambda i: (0, 0)),
        ],
        out_specs=[
            pl.BlockSpec((_TCR, 512), lambda i: (i, 0)),
            pl.BlockSpec((_TCR, 16), lambda i: (i, 0)),
        ],
        out_shape=[
            jax.ShapeDtypeStruct((N, 512), _f32),
            jax.ShapeDtypeStruct((N, 16), _f32),
        ],
    )(acc1p, den1p, b1.reshape(1, 128), W2, A2, Exp1)


def _out_body(acc_ref, den_ref, exp2_ref, m_ref, b2_ref, out_ref):
    a = acc_ref[0] + acc_ref[1]
    val = jnp.concatenate([a[0], a[1], a[2], a[3]], axis=-1)
    dens = den_ref[0] + den_ref[1]
    denx = jnp.dot(dens, exp2_ref[...], preferred_element_type=_f32)
    val = val / (denx + 1e-16)
    z = jnp.dot(val, m_ref[...], preferred_element_type=_f32) + b2_ref[...]
    zm = z - jnp.max(z, axis=-1, keepdims=True)
    out_ref[...] = zm - jnp.log(jnp.sum(jnp.exp(zm), axis=-1, keepdims=True))


def _tc_out(acc2p, den2p, Exp2, M, b2):
    return pl.pallas_call(
        _out_body,
        grid=(_TCG,),
        in_specs=[
            pl.BlockSpec((2, 4, _TCR, 128), lambda i: (0, 0, i, 0)),
            pl.BlockSpec((2, _TCR, 16), lambda i: (0, i, 0)),
            pl.BlockSpec((16, 512), lambda i: (0, 0)),
            pl.BlockSpec((512, 64), lambda i: (0, 0)),
            pl.BlockSpec((1, 64), lambda i: (0, 0)),
        ],
        out_specs=pl.BlockSpec((_TCR, 64), lambda i: (i, 0)),
        out_shape=jax.ShapeDtypeStruct((N, 64), _f32),
    )(acc2p, den2p, Exp2, M, b2.reshape(1, 64))


def _mesh():
    return plsc.VectorSubcoreMesh(
        core_axis_name="c", subcore_axis_name="s",
        num_cores=NCORE, num_subcores=NSUB)


def _zero_vmem(ref, rows, width):
    z = jnp.zeros((16,), _f32)

    def body(r, _):
        for j in range(width // 16):
            ref[r, pl.ds(16 * j, 16)] = z
        return 0
    lax.fori_loop(0, rows, body, 0)


def _splat(ref2d, k, lane_idx):
    row = jnp.full((16,), k, jnp.int32)
    return plsc.load_gather(ref2d, [row, lane_idx])


def _scores(asrc_v, adst_v, ex_v, lane, shift_idx):
    def body(k, _):
        a = asrc_v[k, :]
        b = plsc.load_gather(adst_v, [jnp.full((16,), k, jnp.int32), shift_idx])
        e = a + b
        e = jnp.where(e >= 0.0, e, e * jnp.float32(0.2))
        ex_v[k, :] = jnp.where(lane < 8, jnp.exp(e), jnp.float32(0.0))
        return 0
    lax.fori_loop(0, CK, body, 0)


def _l1_body(block_ref, asad_ref, xw_ref, accp_ref, denp_ref,
             src_v, dst_v, asrc_v, adst_v, ex_v, xw_v, zb_acc, zb_den, bb,
             acc_sh, den_sh, sem_a, sem_b, sem_x):
    c = lax.axis_index("c")
    s = lax.axis_index("s")
    lane = lax.iota(jnp.int32, 16)
    shift_idx = lane % 8 + 8
    row0 = s * RPT

    _zero_vmem(zb_acc, RB, 128)
    _zero_vmem(zb_den, RPT, 16)
    for j in range(5):
        pltpu.sync_copy(zb_acc, acc_sh.at[pl.ds(row0 + j * RB, RB)])
    pltpu.sync_copy(zb_den, den_sh.at[pl.ds(row0, RPT)])
    plsc.subcore_barrier()

    base = c * E_PER_SC + s * E_PER_TILE

    def chunk(ch, _):
        off = base + ch * CK
        pltpu.sync_copy(block_ref.at[0, pl.ds(off, CK)], src_v)
        pltpu.sync_copy(block_ref.at[1, pl.ds(off, CK)], dst_v)
        cp_a = pltpu.async_copy(asad_ref.at[src_v], asrc_v, sem_a)
        cp_b = pltpu.async_copy(asad_ref.at[dst_v], adst_v, sem_b)
        cp_x = pltpu.async_copy(xw_ref.at[src_v], xw_v, sem_x)
        cp_a.wait()
        cp_b.wait()
        _scores(asrc_v, adst_v, ex_v, lane, shift_idx)
        pltpu.sync_copy(ex_v, den_sh.at[dst_v], add=True)
        cp_x.wait()

        def mul(k, _):
            for h in range(H):
                sp = _splat(ex_v, k, jnp.full((16,), h, jnp.int32))
                xw_v[k, pl.ds(16 * h, 16)] = xw_v[k, pl.ds(16 * h, 16)] * sp
            return 0
        lax.fori_loop(0, CK, mul, 0)
        pltpu.sync_copy(xw_v, acc_sh.at[dst_v], add=True)
        return 0

    lax.fori_loop(0, NCHUNK, chunk, 0)
    plsc.subcore_barrier()

    for j in range(5):
        pltpu.sync_copy(acc_sh.at[pl.ds(row0 + j * RB, RB)], bb)
        pltpu.sync_copy(bb, accp_ref.at[c, pl.ds(row0 + j * RB, RB)])
    pltpu.sync_copy(den_sh.at[pl.ds(row0, RPT)], zb_den)
    pltpu.sync_copy(zb_den, denp_ref.at[c, pl.ds(row0, RPT)])


def _sc_layer1(block, asad1, xw1):
    kfn = pl.kernel(
        _l1_body,
        out_type=(
            jax.ShapeDtypeStruct((NCORE, N, 128), _f32),
            jax.ShapeDtypeStruct((NCORE, N, 16), _f32),
        ),
        mesh=_mesh(),
        scratch_types=[
            pltpu.VMEM((CK,), jnp.int32),
            pltpu.VMEM((CK,), jnp.int32),
            pltpu.VMEM((CK, 16), _f32),
            pltpu.VMEM((CK, 16), _f32),
            pltpu.VMEM((CK, 16), _f32),
            pltpu.VMEM((CK, 128), _f32),
            pltpu.VMEM((RB, 128), _f32),
            pltpu.VMEM((RPT, 16), _f32),
            pltpu.VMEM((RB, 128), _f32),
            pltpu.VMEM_SHARED((N, 128), _f32),
            pltpu.VMEM_SHARED((N, 16), _f32),
            pltpu.SemaphoreType.DMA,
            pltpu.SemaphoreType.DMA,
            pltpu.SemaphoreType.DMA,
        ],
    )
    return kfn(block, asad1, xw1)


def _l2_body(block_ref, asad_ref, xw2v_ref, accp_ref, denp_ref, ex2_ref,
             src_v, dst_v, idx_v, asrc_v, adst_v, ex_v, xw_v,
             zb_acc, zb_den, bb, acc_sh, den_sh, sem_a, sem_b, sem_x):
    c = lax.axis_index("c")
    s = lax.axis_index("s")
    lane = lax.iota(jnp.int32, 16)
    shift_idx = lane % 8 + 8
    row0 = s * RPT
    base = c * E_PER_SC + s * E_PER_TILE

    _zero_vmem(zb_acc, RB, 128)
    _zero_vmem(zb_den, RPT, 16)
    pltpu.sync_copy(zb_den, den_sh.at[pl.ds(row0, RPT)])
    plsc.subcore_barrier()

    def chunk_a(ch, _):
        off = base + ch * CK
        pltpu.sync_copy(block_ref.at[0, pl.ds(off, CK)], src_v)
        pltpu.sync_copy(block_ref.at[1, pl.ds(off, CK)], dst_v)
        cp_a = pltpu.async_copy(asad_ref.at[src_v], asrc_v, sem_a)
        cp_b = pltpu.async_copy(asad_ref.at[dst_v], adst_v, sem_b)
        cp_a.wait()
        cp_b.wait()
        _scores(asrc_v, adst_v, ex_v, lane, shift_idx)
        pltpu.sync_copy(ex_v, den_sh.at[dst_v], add=True)
        pltpu.sync_copy(ex_v, ex2_ref.at[pl.ds(off, CK)])
        return 0

    lax.fori_loop(0, NCHUNK, chunk_a, 0)
    plsc.subcore_barrier()
    pltpu.sync_copy(den_sh.at[pl.ds(row0, RPT)], zb_den)
    pltpu.sync_copy(zb_den, denp_ref.at[c, pl.ds(row0, RPT)])
    _zero_vmem(zb_den, RPT, 16)

    for hp in range(4):
        for j in range(5):
            pltpu.sync_copy(zb_acc, acc_sh.at[pl.ds(row0 + j * RB, RB)])
        plsc.subcore_barrier()

        def chunk_b(ch, _):
            off = base + ch * CK
            pltpu.sync_copy(block_ref.at[0, pl.ds(off, CK)], src_v)
            pltpu.sync_copy(block_ref.at[1, pl.ds(off, CK)], dst_v)
            for i in range(CK // 16):
                v = src_v[pl.ds(16 * i, 16)]
                idx_v[pl.ds(16 * i, 16)] = v * 4 + hp
            pltpu.sync_copy(ex2_ref.at[pl.ds(off, CK)], ex_v)
            cp_x = pltpu.async_copy(xw2v_ref.at[idx_v], xw_v, sem_x)
            cp_x.wait()

            def mul(k, _):
                for j in range(8):
                    ln = hp * 2 + j // 4
                    sp = _splat(ex_v, k, jnp.full((16,), ln, jnp.int32))
                    xw_v[k, pl.ds(16 * j, 16)] = xw_v[k, pl.ds(16 * j, 16)] * sp
                return 0
            lax.fori_loop(0, CK, mul, 0)
            pltpu.sync_copy(xw_v, acc_sh.at[dst_v], add=True)
            return 0

        lax.fori_loop(0, NCHUNK, chunk_b, 0)
        plsc.subcore_barrier()
        for j in range(5):
            pltpu.sync_copy(acc_sh.at[pl.ds(row0 + j * RB, RB)], bb)
            pltpu.sync_copy(bb, accp_ref.at[c, hp, pl.ds(row0 + j * RB, RB)])


def _sc_layer2(block, asad2, xw2v):
    kfn = pl.kernel(
        _l2_body,
        out_type=(
            jax.ShapeDtypeStruct((NCORE, 4, N, 128), _f32),
            jax.ShapeDtypeStruct((NCORE, N, 16), _f32),
            jax.ShapeDtypeStruct((E, 16), _f32),
        ),
        mesh=_mesh(),
        scratch_types=[
            pltpu.VMEM((CK,), jnp.int32),
            pltpu.VMEM((CK,), jnp.int32),
            pltpu.VMEM((CK,), jnp.int32),
            pltpu.VMEM((CK, 16), _f32),
            pltpu.VMEM((CK, 16), _f32),
            pltpu.VMEM((CK, 16), _f32),
            pltpu.VMEM((CK, 128), _f32),
            pltpu.VMEM((RB, 128), _f32),
            pltpu.VMEM((RPT, 16), _f32),
            pltpu.VMEM((RB, 128), _f32),
            pltpu.VMEM_SHARED((N, 128), _f32),
            pltpu.VMEM_SHARED((N, 16), _f32),
            pltpu.SemaphoreType.DMA,
            pltpu.SemaphoreType.DMA,
            pltpu.SemaphoreType.DMA,
        ],
    )
    return kfn(block, asad2, xw2v)


def _blockdiag(att_src, att_dst):
    h, cdim = att_src.shape
    eye = jnp.eye(h, dtype=_f32)
    asrc = (att_src[:, :, None] * eye[:, None, :]).reshape(h * cdim, h)
    adst = (att_dst[:, :, None] * eye[:, None, :]).reshape(h * cdim, h)
    return jnp.concatenate([asrc, adst], axis=1)


def _expand_mat(cdim):
    top = jnp.repeat(jnp.eye(H, dtype=_f32), cdim, axis=1)
    return jnp.concatenate([top, jnp.zeros_like(top)], axis=0)


def kernel(x, block, W1, att_src1, att_dst1, b1, W2, att_src2, att_dst2, b2):
    A1 = _blockdiag(att_src1, att_dst1)
    A2 = _blockdiag(att_src2, att_dst2)
    Exp1 = _expand_mat(C1)
    Exp2 = _expand_mat(C2)
    M = jnp.tile(jnp.eye(C2, dtype=_f32), (H, 1)) * (1.0 / H)

    xw1, asad1 = _tc_mm_score(x, W1, A1)
    acc1p, den1p = _sc_layer1(block, asad1, xw1)
    xw2, asad2 = _tc_mid(acc1p, den1p, b1, W2, A2, Exp1)
    acc2p, den2p, _ = _sc_layer2(block, asad2, xw2.reshape(N * 4, 128))
    return _tc_out(acc2p, den2p, Exp2, M, b2)


# speedup vs baseline: 17.2851x; 17.2851x over previous
import jax
import jax.numpy as jnp
from jax import lax
from jax.experimental import pallas as pl
from jax.experimental.pallas import tpu as pltpu
from jax.experimental.pallas import tpu_sc as plsc

N = 10000
E = 320000
H = 8
C1 = 16
C2 = 64
NCORE = 2
NSUB = 16
LANES = 16
E_PER_SC = E // NCORE
E_PER_TILE = E_PER_SC // NSUB
CK = 80
NCHUNK = E_PER_TILE // CK
NP = 10240
RPT = NP // NSUB
RB = 128

_f32 = jnp.float32


_TCR = 400
_TCG = N // _TCR


def _mm_score_body(x_ref, w_ref, a_ref, xw_ref, sc_ref):
    xw = jnp.dot(x_ref[...], w_ref[...], preferred_element_type=_f32)
    xw_ref[...] = xw
    sc_ref[...] = jnp.dot(xw, a_ref[...], preferred_element_type=_f32)


def _tc_mm_score(x, W, A):
    f_in = x.shape[1]
    f_out = W.shape[1]
    return pl.pallas_call(
        _mm_score_body,
        grid=(_TCG,),
        in_specs=[
            pl.BlockSpec((_TCR, f_in), lambda i: (i, 0)),
            pl.BlockSpec((f_in, f_out), lambda i: (0, 0)),
            pl.BlockSpec((f_out, 16), lambda i: (0, 0)),
        ],
        out_specs=[
            pl.BlockSpec((_TCR, f_out), lambda i: (i, 0)),
            pl.BlockSpec((_TCR, 16), lambda i: (i, 0)),
        ],
        out_shape=[
            jax.ShapeDtypeStruct((N, f_out), _f32),
            jax.ShapeDtypeStruct((N, 16), _f32),
        ],
    )(x, W, A)


def _mid_body(acc_ref, den_ref, b1_ref, w2_ref, a2_ref, exp1_ref,
              xw2_ref, sc2_ref):
    a = acc_ref[0] + acc_ref[1]
    accs = jnp.concatenate([a[0], a[1]], axis=-1)
    dens = den_ref[0] + den_ref[1]
    denx = jnp.dot(dens, exp1_ref[...], preferred_element_type=_f32)
    h1 = accs / (denx + 1e-16) + b1_ref[...]
    h1 = jnp.where(h1 > 0, h1, jnp.exp(jnp.minimum(h1, 0.0)) - 1.0)
    xw2 = jnp.dot(h1, w2_ref[...], preferred_element_type=_f32)
    xw2_ref[...] = xw2
    sc2_ref[...] = jnp.dot(xw2, a2_ref[...], preferred_element_type=_f32)


def _tc_mid(acc1p, den1p, b1, W2, A2, Exp1):
    return pl.pallas_call(
        _mid_body,
        grid=(_TCG,),
        in_specs=[
            pl.BlockSpec((2, 2, _TCR, 64), lambda i: (0, 0, i, 0)),
            pl.BlockSpec((2, _TCR, 16), lambda i: (0, i, 0)),
            pl.BlockSpec((1, 128), lambda i: (0, 0)),
            pl.BlockSpec((128, 512), lambda i: (0, 0)),
            pl.BlockSpec((512, 16), lambda i: (0, 0)),
            pl.BlockSpec((16, 128), lambda i: (0, 0)),
        ],
        out_specs=[
            pl.BlockSpec((_TCR, 512), lambda i: (i, 0)),
            pl.BlockSpec((_TCR, 16), lambda i: (i, 0)),
        ],
        out_shape=[
            jax.ShapeDtypeStruct((N, 512), _f32),
            jax.ShapeDtypeStruct((N, 16), _f32),
        ],
    )(acc1p, den1p, b1.reshape(1, 128), W2, A2, Exp1)


def _out_body(acc_ref, den_ref, exp2_ref, m_ref, b2_ref, out_ref):
    a = acc_ref[0] + acc_ref[1]
    val = jnp.concatenate([a[i] for i in range(8)], axis=-1)
    dens = den_ref[0] + den_ref[1]
    denx = jnp.dot(dens, exp2_ref[...], preferred_element_type=_f32)
    val = val / (denx + 1e-16)
    z = jnp.dot(val, m_ref[...], preferred_element_type=_f32) + b2_ref[...]
    zm = z - jnp.max(z, axis=-1, keepdims=True)
    out_ref[...] = zm - jnp.log(jnp.sum(jnp.exp(zm), axis=-1, keepdims=True))


def _tc_out(acc2p, den2p, Exp2, M, b2):
    return pl.pallas_call(
        _out_body,
        grid=(_TCG,),
        in_specs=[
            pl.BlockSpec((2, 8, _TCR, 64), lambda i: (0, 0, i, 0)),
            pl.BlockSpec((2, _TCR, 16), lambda i: (0, i, 0)),
            pl.BlockSpec((16, 512), lambda i: (0, 0)),
            pl.BlockSpec((512, 64), lambda i: (0, 0)),
            pl.BlockSpec((1, 64), lambda i: (0, 0)),
        ],
        out_specs=pl.BlockSpec((_TCR, 64), lambda i: (i, 0)),
        out_shape=jax.ShapeDtypeStruct((N, 64), _f32),
    )(acc2p, den2p, Exp2, M, b2.reshape(1, 64))


def _mesh():
    return plsc.VectorSubcoreMesh(
        core_axis_name="c", subcore_axis_name="s",
        num_cores=NCORE, num_subcores=NSUB)


def _zero_vmem(ref, rows, width):
    z = jnp.zeros((16,), _f32)

    def body(r, _):
        for j in range(width // 16):
            ref[r, pl.ds(16 * j, 16)] = z
        return 0
    lax.fori_loop(0, rows, body, 0)


_GDN = lax.GatherDimensionNumbers(
    offset_dims=(), collapsed_slice_dims=(0,), start_index_map=(0,))


def _permute(vec, idx):
    return lax.gather(vec, idx[:, None], _GDN, (1,),
                      mode=lax.GatherScatterMode.PROMISE_IN_BOUNDS)


def _scores(asrc_v, adst_v, ex_v, lane, shift_idx):
    def body(k, _):
        a = asrc_v[k, :]
        b = _permute(adst_v[k, :], shift_idx)
        e = a + b
        e = jnp.where(e >= 0.0, e, e * jnp.float32(0.2))
        ex_v[k, :] = jnp.where(lane < 8, jnp.exp(e), jnp.float32(0.0))
        return 0
    lax.fori_loop(0, CK, body, 0)


def _make_sc_body(n_sweeps, c_h):
    def body(src_ref, dst_ref, asad_ref, tab_ref, accp_ref, denp_ref, exs_ref,
             src_v, dst_v, idx_v, asrc_v, adst_v, ex_v, row_v,
             zb, zb_den, bb, acc_sh, den_sh, sem_a, sem_b, sem_x):
        c = lax.axis_index("c")
        s = lax.axis_index("s")
        lane = lax.iota(jnp.int32, 16)
        shift_idx = lane % 8 + 8
        row0 = s * RPT
        base = c * E_PER_SC + s * E_PER_TILE

        _zero_vmem(zb, RB, 64)
        _zero_vmem(zb_den, RPT, 16)
        pltpu.sync_copy(zb_den, den_sh.at[pl.ds(row0, RPT)])
        plsc.subcore_barrier()

        def chunk_a(ch, _):
            off = base + ch * CK
            pltpu.sync_copy(src_ref.at[pl.ds(off, CK)], src_v)
            pltpu.sync_copy(dst_ref.at[pl.ds(off, CK)], dst_v)
            cp_a = pltpu.async_copy(asad_ref.at[src_v], asrc_v, sem_a)
            cp_b = pltpu.async_copy(asad_ref.at[dst_v], adst_v, sem_b)
            cp_a.wait()
            cp_b.wait()
            _scores(asrc_v, adst_v, ex_v, lane, shift_idx)
            pltpu.sync_copy(ex_v, den_sh.at[dst_v], add=True)
            pltpu.sync_copy(ex_v, exs_ref.at[pl.ds(off, CK)])
            return 0

        lax.fori_loop(0, NCHUNK, chunk_a, 0)
        plsc.subcore_barrier()
        pltpu.sync_copy(den_sh.at[pl.ds(row0, RPT)], zb_den)
        pltpu.sync_copy(zb_den, denp_ref.at[c, pl.ds(row0, RPT)])

        for sw in range(n_sweeps):
            for j in range(5):
                pltpu.sync_copy(zb, acc_sh.at[pl.ds(row0 + j * RB, RB)])
            plsc.subcore_barrier()

            def chunk_b(ch, _, sw=sw):
                off = base + ch * CK
                pltpu.sync_copy(src_ref.at[pl.ds(off, CK)], src_v)
                pltpu.sync_copy(dst_ref.at[pl.ds(off, CK)], dst_v)
                for i in range(CK // 16):
                    v = src_v[pl.ds(16 * i, 16)]
                    idx_v[pl.ds(16 * i, 16)] = v * n_sweeps + sw
                pltpu.sync_copy(exs_ref.at[pl.ds(off, CK)], ex_v)
                cp_x = pltpu.async_copy(tab_ref.at[idx_v], row_v, sem_x)
                cp_x.wait()

                def mul(k, _):
                    exv = ex_v[k, :]
                    for j in range(4):
                        h = (sw * 64 + 16 * j) // c_h
                        sp = _permute(exv, jnp.full((16,), h, jnp.int32))
                        row_v[k, pl.ds(16 * j, 16)] = (
                            row_v[k, pl.ds(16 * j, 16)] * sp)
                    return 0
                lax.fori_loop(0, CK, mul, 0)
                pltpu.sync_copy(row_v, acc_sh.at[dst_v], add=True)
                return 0

            lax.fori_loop(0, NCHUNK, chunk_b, 0)
            plsc.subcore_barrier()
            for j in range(5):
                pltpu.sync_copy(acc_sh.at[pl.ds(row0 + j * RB, RB)], bb)
                pltpu.sync_copy(bb, accp_ref.at[c, sw, pl.ds(row0 + j * RB, RB)])

    return body


def _sc_edge(src, dst, asad, tab, n_sweeps, c_h):
    kfn = pl.kernel(
        _make_sc_body(n_sweeps, c_h),
        out_type=(
            jax.ShapeDtypeStruct((NCORE, n_sweeps, NP, 64), _f32),
            jax.ShapeDtypeStruct((NCORE, NP, 16), _f32),
            jax.ShapeDtypeStruct((E, 16), _f32),
        ),
        mesh=_mesh(),
        compiler_params=pltpu.CompilerParams(use_tc_tiling_on_sc=False),
        scratch_types=[
            pltpu.VMEM((CK,), jnp.int32),
            pltpu.VMEM((CK,), jnp.int32),
            pltpu.VMEM((CK,), jnp.int32),
            pltpu.VMEM((CK, 16), _f32),
            pltpu.VMEM((CK, 16), _f32),
            pltpu.VMEM((CK, 16), _f32),
            pltpu.VMEM((CK, 64), _f32),
            pltpu.VMEM((RB, 64), _f32),
            pltpu.VMEM((RPT, 16), _f32),
            pltpu.VMEM((RB, 64), _f32),
            pltpu.VMEM_SHARED((NP, 64), _f32),
            pltpu.VMEM_SHARED((NP, 16), _f32),
            pltpu.SemaphoreType.DMA,
            pltpu.SemaphoreType.DMA,
            pltpu.SemaphoreType.DMA,
        ],
    )
    return kfn(src, dst, asad, tab)


def _blockdiag(att_src, att_dst):
    h, cdim = att_src.shape
    eye = jnp.eye(h, dtype=_f32)
    asrc = (att_src[:, :, None] * eye[:, None, :]).reshape(h * cdim, h)
    adst = (att_dst[:, :, None] * eye[:, None, :]).reshape(h * cdim, h)
    return jnp.concatenate([asrc, adst], axis=1)


def _expand_mat(cdim):
    top = jnp.repeat(jnp.eye(H, dtype=_f32), cdim, axis=1)
    return jnp.concatenate([top, jnp.zeros_like(top)], axis=0)


def kernel(x, block, W1, att_src1, att_dst1, b1, W2, att_src2, att_dst2, b2):
    A1 = _blockdiag(att_src1, att_dst1)
    A2 = _blockdiag(att_src2, att_dst2)
    Exp1 = _expand_mat(C1)
    Exp2 = _expand_mat(C2)
    M = jnp.tile(jnp.eye(C2, dtype=_f32), (H, 1)) * (1.0 / H)

    src = block[0]
    dst = block[1]
    xw1, asad1 = _tc_mm_score(x, W1, A1)
    acc1p, den1p, _ = _sc_edge(src, dst, asad1, xw1.reshape(N * 2, 64), 2, C1)
    xw2, asad2 = _tc_mid(acc1p, den1p, b1, W2, A2, Exp1)
    acc2p, den2p, _ = _sc_edge(src, dst, asad2, xw2.reshape(N * 8, 64), 8, C2)
    return _tc_out(acc2p, den2p, Exp2, M, b2)
